# baseline (speedup 1.0000x reference)
import functools

import jax
import jax.numpy as jnp
from jax import lax
from jax.experimental import pallas as pl
from jax.experimental.pallas import tpu as pltpu
from jax.experimental.pallas import tpu_sc as plsc

L = 16


def kernel(node_ids, table):
    (B,) = node_ids.shape
    V, D = table.shape
    NW = 32
    NTC = (V + 127) // 128
    TPW = (NTC + NW - 1) // NW
    K = 5
    CW = K * 128
    NCH = (TPW + K - 1) // K
    PHYS = NTC * 128
    G = 32
    NG = B // L

    tableT = table.T

    @functools.partial(
        pl.kernel,
        mesh=plsc.VectorSubcoreMesh(core_axis_name="c", subcore_axis_name="s"),
        out_type=jax.ShapeDtypeStruct((B + NW * G, 128), jnp.float32),
        scratch_types=[
            pltpu.VMEM((B,), jnp.int32),
            pltpu.VMEM((4 * (B // 4 + L),), jnp.int32),
            pltpu.VMEM((D, CW), jnp.float32),
            pltpu.VMEM((D, CW), jnp.float32),
            pltpu.VMEM((G, 128), jnp.float32),
            pltpu.VMEM((G + L,), jnp.int32),
            pltpu.VMEM((G + L,), jnp.int32),
            pltpu.VMEM((G,), jnp.int32),
            pltpu.SemaphoreType.DMA,
            pltpu.SemaphoreType.DMA,
            pltpu.SemaphoreType.DMA,
        ],
        compiler_params=pltpu.CompilerParams(needs_layout_passes=False),
    )
    def emb(tT_hbm, ids_hbm, out_hbm, ids_v, blist, stage0, stage1, rowbuf,
            cb, co, bidx, sem_s0, sem_s1, sem_w):
        wid = lax.axis_index("s") * 2 + lax.axis_index("c")
        dummy0 = B + wid * G
        lo = wid * (TPW * 128)
        hi = jnp.minimum(lo + TPW * 128, V)
        lane = lax.iota(jnp.int32, L)

        pltpu.sync_copy(ids_hbm, ids_v)

        last_lane = jnp.full((L,), L - 1, jnp.int32)

        dnums = lax.GatherDimensionNumbers(
            offset_dims=(), collapsed_slice_dims=(0,), start_index_map=(0,))

        def splat_last(x):
            return lax.gather(
                x, last_lane[:, None], dimension_numbers=dnums,
                slice_sizes=(1,),
                mode=lax.GatherScatterMode.PROMISE_IN_BOUNDS)

        NS4 = 4
        CAPG = NG // NS4
        SEG = B // NS4 + L

        def scan_body(i, st):
            outs = []
            for ss in range(NS4):
                g = i + ss * CAPG
                v = ids_v[pl.ds(pl.multiple_of(g * L, L), L)]
                b_vec = lane + g * L
                m = (v >= lo) & (v < hi)
                cs = plsc.cumsum(m.astype(jnp.int32))
                pos = (ss * SEG) + st[ss] + cs - 1
                plsc.store_scatter(blist, [pos], b_vec, mask=m)
                outs.append(st[ss] + splat_last(cs))
            return tuple(outs)

        z16 = jnp.zeros((L,), jnp.int32)
        cnts = lax.fori_loop(0, CAPG, scan_body, (z16, z16, z16, z16),
                             unroll=False)
        ngrs = [(jnp.max(cv) + L - 1) // L for cv in cnts]
        cnt_of = [jnp.max(cv) for cv in cnts]

        for i in range(G // L):
            bidx[pl.ds(i * L, L)] = dummy0 + lane + i * L

        def drain_scatter():
            pltpu.make_async_copy(rowbuf, out_hbm.at[bidx], sem_w).wait()

        pltpu.async_copy(rowbuf, out_hbm.at[bidx], sem_w)

        def fire(q, src):
            drain_scatter()
            for i in range(G // L):
                bidx[pl.ds(i * L, L)] = dummy0 + lane + i * L
            for sg in range(G // L):

                @pl.when(sg * L < q)
                def _():
                    o16 = co[pl.ds(sg * L, L)]
                    b16 = cb[pl.ds(sg * L, L)]
                    vm = (lane + sg * L) < q
                    rows = lane + sg * L

                    def d_body(d, _):
                        dsp = jnp.full((L,), 1, jnp.int32) * d
                        vals = plsc.load_gather(src, [dsp, o16], mask=vm)
                        plsc.store_scatter(rowbuf, [rows, dsp], vals, mask=vm)
                        return 0

                    lax.fori_loop(0, D, d_body, 0, unroll=8)
                    plsc.store_scatter(bidx, [rows], b16, mask=vm)

            pltpu.async_copy(rowbuf, out_hbm.at[bidx], sem_w)

        def sweep(c0, c1, sbase, src):
            q = jnp.int32(0)
            for ss in range(NS4):

                def rs_cond(st):
                    gi, q = st
                    return gi < ngrs[ss]

                def rs_body(st):
                    gi, q = st
                    b16 = blist[pl.ds(ss * SEG + pl.multiple_of(gi * L, L), L)]
                    vm = (lane + gi * L) < cnt_of[ss]
                    hid = plsc.load_gather(ids_v, [b16], mask=vm)
                    m = vm & (hid >= c0) & (hid < c1)
                    cs = plsc.cumsum(m.astype(jnp.int32))
                    pos = q + cs - 1
                    plsc.store_scatter(cb, [pos], b16, mask=m)
                    plsc.store_scatter(co, [pos], hid - sbase, mask=m)
                    q = q + jnp.max(cs)

                    @pl.when(q > G - L)
                    def _():
                        fire(q, src)

                    return gi + 1, jnp.where(q > G - L, 0, q)

                _, q = lax.while_loop(rs_cond, rs_body, (jnp.int32(0), q))

            @pl.when(q > 0)
            def _():
                fire(q, src)

        def sbase_of(c):
            c0 = lo + c * CW
            return pl.multiple_of(jnp.minimum(c0, PHYS - CW), 128)

        def issue(c, stg, sem):
            s = sbase_of(c)
            for i in range(D // 8):
                pltpu.async_copy(
                    tT_hbm.at[pl.ds(8 * i, 8), pl.ds(s, CW)],
                    stg.at[pl.ds(8 * i, 8), :],
                    sem,
                )

        def drain_stage(stg, sem):
            for i in range(D // 8):
                pltpu.make_async_copy(
                    tT_hbm.at[pl.ds(0, 8), pl.ds(0, CW)],
                    stg.at[pl.ds(8 * i, 8), :],
                    sem,
                ).wait()

        def do_chunk(c, stg):
            c0 = lo + c * CW
            c1 = jnp.minimum(c0 + CW, hi)
            sweep(c0, c1, sbase_of(c), stg)

        issue(0, stage0, sem_s0)

        def pair_body(p, _):
            c = p * 2
            drain_stage(stage0, sem_s0)

            @pl.when(c + 1 < NCH)
            def _():
                issue(c + 1, stage1, sem_s1)

            do_chunk(c, stage0)

            @pl.when(c + 1 < NCH)
            def _():
                drain_stage(stage1, sem_s1)

                @pl.when(c + 2 < NCH)
                def _():
                    issue(c + 2, stage0, sem_s0)

                do_chunk(c + 1, stage1)

            return 0

        lax.fori_loop(0, (NCH + 1) // 2, pair_body, 0, unroll=False)
        drain_scatter()

    out_full = emb(tableT, node_ids.astype(jnp.int32))
    return out_full[:B, :D]


# speedup vs baseline: 1.3159x; 1.3159x over previous
import functools

import jax
import jax.numpy as jnp
from jax import lax
from jax.experimental import pallas as pl
from jax.experimental.pallas import tpu as pltpu
from jax.experimental.pallas import tpu_sc as plsc

L = 16


def kernel(node_ids, table):
    (B,) = node_ids.shape
    V, D = table.shape
    NW = 32
    NTC = (V + 127) // 128
    TPW = (NTC + NW - 1) // NW
    K = 5
    CW = K * 128
    NCH = (TPW + K - 1) // K
    PHYS = NTC * 128
    G = 32
    NG = B // L

    tableT = table.T

    @functools.partial(
        pl.kernel,
        mesh=plsc.VectorSubcoreMesh(core_axis_name="c", subcore_axis_name="s"),
        out_type=jax.ShapeDtypeStruct((B + NW * G, 128), jnp.float32),
        scratch_types=[
            pltpu.VMEM((B,), jnp.int32),
            pltpu.VMEM((B + L,), jnp.int32),
            pltpu.VMEM((D, CW), jnp.float32),
            pltpu.VMEM((D, CW), jnp.float32),
            pltpu.VMEM((G, 128), jnp.float32),
            pltpu.VMEM((G + L,), jnp.int32),
            pltpu.VMEM((G + L,), jnp.int32),
            pltpu.VMEM((G,), jnp.int32),
            pltpu.SemaphoreType.DMA,
            pltpu.SemaphoreType.DMA,
            pltpu.SemaphoreType.DMA,
        ],
        compiler_params=pltpu.CompilerParams(needs_layout_passes=False),
    )
    def emb(tT_hbm, ids_hbm, out_hbm, ids_v, blist, stage0, stage1, rowbuf,
            cb, co, bidx, sem_s0, sem_s1, sem_w):
        wid = lax.axis_index("s") * 2 + lax.axis_index("c")
        dummy0 = B + wid * G
        lo = wid * (TPW * 128)
        hi = jnp.minimum(lo + TPW * 128, V)
        lane = lax.iota(jnp.int32, L)

        pltpu.sync_copy(ids_hbm, ids_v)

        def scan_body(i, cnt):
            v = ids_v[pl.ds(pl.multiple_of(i * L, L), L)]
            b_vec = lane + i * L
            m = (v >= lo) & (v < hi)
            mi = m.astype(jnp.int32)
            pos = cnt + plsc.cumsum(mi) - mi
            plsc.store_scatter(blist, [pos], b_vec, mask=m)
            return cnt + jnp.sum(mi)

        cnt = lax.fori_loop(0, NG, scan_body, jnp.int32(0), unroll=False)
        ngr = (cnt + L - 1) // L

        for i in range(G // L):
            bidx[pl.ds(i * L, L)] = dummy0 + lane + i * L

        def drain_scatter():
            pltpu.make_async_copy(rowbuf, out_hbm.at[bidx], sem_w).wait()

        pltpu.async_copy(rowbuf, out_hbm.at[bidx], sem_w)

        def fire(q, src):
            drain_scatter()
            for i in range(G // L):
                bidx[pl.ds(i * L, L)] = dummy0 + lane + i * L
            for sg in range(G // L):

                @pl.when(sg * L < q)
                def _():
                    o16 = co[pl.ds(sg * L, L)]
                    b16 = cb[pl.ds(sg * L, L)]
                    vm = (lane + sg * L) < q
                    rows = lane + sg * L

                    def d_body(d, _):
                        dsp = jnp.full((L,), 1, jnp.int32) * d
                        vals = plsc.load_gather(src, [dsp, o16], mask=vm)
                        plsc.store_scatter(rowbuf, [rows, dsp], vals, mask=vm)
                        return 0

                    lax.fori_loop(0, D, d_body, 0, unroll=8)
                    plsc.store_scatter(bidx, [rows], b16, mask=vm)

            pltpu.async_copy(rowbuf, out_hbm.at[bidx], sem_w)

        def sweep(c0, c1, sbase, src):
            def rs_cond(st):
                gi, q = st
                return gi < ngr

            def rs_body(st):
                gi, q = st
                b16 = blist[pl.ds(pl.multiple_of(gi * L, L), L)]
                vm = (lane + gi * L) < cnt
                hid = plsc.load_gather(ids_v, [b16], mask=vm)
                m = vm & (hid >= c0) & (hid < c1)
                mi = m.astype(jnp.int32)
                pos = q + plsc.cumsum(mi) - mi
                plsc.store_scatter(cb, [pos], b16, mask=m)
                plsc.store_scatter(co, [pos], hid - sbase, mask=m)
                q = q + jnp.sum(mi)

                @pl.when(q > G - L)
                def _():
                    fire(q, src)

                return gi + 1, jnp.where(q > G - L, 0, q)

            gi, q = lax.while_loop(rs_cond, rs_body, (jnp.int32(0), jnp.int32(0)))

            @pl.when(q > 0)
            def _():
                fire(q, src)

        def sbase_of(c):
            c0 = lo + c * CW
            return pl.multiple_of(jnp.minimum(c0, PHYS - CW), 128)

        def issue(c, stg, sem):
            s = sbase_of(c)
            for i in range(D // 8):
                pltpu.async_copy(
                    tT_hbm.at[pl.ds(8 * i, 8), pl.ds(s, CW)],
                    stg.at[pl.ds(8 * i, 8), :],
                    sem,
                )

        def drain_stage(stg, sem):
            for i in range(D // 8):
                pltpu.make_async_copy(
                    tT_hbm.at[pl.ds(0, 8), pl.ds(0, CW)],
                    stg.at[pl.ds(8 * i, 8), :],
                    sem,
                ).wait()

        def do_chunk(c, stg):
            c0 = lo + c * CW
            c1 = jnp.minimum(c0 + CW, hi)
            sweep(c0, c1, sbase_of(c), stg)

        issue(0, stage0, sem_s0)

        def pair_body(p, _):
            c = p * 2
            drain_stage(stage0, sem_s0)

            @pl.when(c + 1 < NCH)
            def _():
                issue(c + 1, stage1, sem_s1)

            do_chunk(c, stage0)

            @pl.when(c + 1 < NCH)
            def _():
                drain_stage(stage1, sem_s1)

                @pl.when(c + 2 < NCH)
                def _():
                    issue(c + 2, stage0, sem_s0)

                do_chunk(c + 1, stage1)

            return 0

        lax.fori_loop(0, (NCH + 1) // 2, pair_body, 0, unroll=False)
        drain_scatter()

    out_full = emb(tableT, node_ids.astype(jnp.int32))
    return out_full[:B, :D]


# speedup vs baseline: 1.3826x; 1.0506x over previous
import functools

import jax
import jax.numpy as jnp
from jax import lax
from jax.experimental import pallas as pl
from jax.experimental.pallas import tpu as pltpu
from jax.experimental.pallas import tpu_sc as plsc

L = 16


def kernel(node_ids, table):
    (B,) = node_ids.shape
    V, D = table.shape
    NW = 32
    NTC = (V + 127) // 128
    TPW = (NTC + NW - 1) // NW
    K = 5
    CW = K * 128
    NCH = (TPW + K - 1) // K
    PHYS = NTC * 128
    G = 32
    NG = B // L

    tableT = table.T

    @functools.partial(
        pl.kernel,
        mesh=plsc.VectorSubcoreMesh(core_axis_name="c", subcore_axis_name="s"),
        out_type=jax.ShapeDtypeStruct((B + NW * G, 128), jnp.float32),
        scratch_types=[
            pltpu.VMEM((B,), jnp.int32),
            pltpu.VMEM((B + L,), jnp.int32),
            pltpu.VMEM((D, CW), jnp.float32),
            pltpu.VMEM((D, CW), jnp.float32),
            pltpu.VMEM((G, 128), jnp.float32),
            pltpu.VMEM((G + L,), jnp.int32),
            pltpu.VMEM((G + L,), jnp.int32),
            pltpu.VMEM((G,), jnp.int32),
            pltpu.SemaphoreType.DMA,
            pltpu.SemaphoreType.DMA,
            pltpu.SemaphoreType.DMA,
        ],
        compiler_params=pltpu.CompilerParams(needs_layout_passes=False),
    )
    def emb(tT_hbm, ids_hbm, out_hbm, ids_v, blist, stage0, stage1, rowbuf,
            cb, co, bidx, sem_s0, sem_s1, sem_w):
        wid = lax.axis_index("s") * 2 + lax.axis_index("c")
        dummy0 = B + wid * G
        lo = wid * (TPW * 128)
        hi = jnp.minimum(lo + TPW * 128, V)
        lane = lax.iota(jnp.int32, L)

        pltpu.sync_copy(ids_hbm, ids_v)

        last_lane = jnp.full((L,), L - 1, jnp.int32)
        dnums = lax.GatherDimensionNumbers(
            offset_dims=(), collapsed_slice_dims=(0,), start_index_map=(0,))

        def splat_last(x):
            return lax.gather(
                x, last_lane[:, None], dimension_numbers=dnums,
                slice_sizes=(1,),
                mode=lax.GatherScatterMode.PROMISE_IN_BOUNDS)

        def scan_body(i, cntv):
            g = i * 2
            v1 = ids_v[pl.ds(pl.multiple_of(g * L, L), L)]
            v2 = ids_v[pl.ds(pl.multiple_of((g + 1) * L, L), L)]
            m1 = (v1 >= lo) & (v1 < hi)
            m2 = (v2 >= lo) & (v2 < hi)
            cs1 = plsc.cumsum(m1.astype(jnp.int32))
            cs2 = plsc.cumsum(m2.astype(jnp.int32))
            plsc.store_scatter(blist, [cntv + cs1 - 1], lane + g * L, mask=m1)
            cntv = cntv + splat_last(cs1)
            plsc.store_scatter(blist, [cntv + cs2 - 1], lane + (g + 1) * L,
                               mask=m2)
            return cntv + splat_last(cs2)

        cntv = lax.fori_loop(0, NG // 2, scan_body, jnp.zeros((L,), jnp.int32),
                             unroll=False)
        cnt = jnp.max(cntv)
        ngr = (cnt + L - 1) // L

        for i in range(G // L):
            bidx[pl.ds(i * L, L)] = dummy0 + lane + i * L

        def drain_scatter():
            pltpu.make_async_copy(rowbuf, out_hbm.at[bidx], sem_w).wait()

        pltpu.async_copy(rowbuf, out_hbm.at[bidx], sem_w)

        def fire(q, src):
            drain_scatter()
            for i in range(G // L):
                bidx[pl.ds(i * L, L)] = dummy0 + lane + i * L
            for sg in range(G // L):

                @pl.when(sg * L < q)
                def _():
                    o16 = co[pl.ds(sg * L, L)]
                    b16 = cb[pl.ds(sg * L, L)]
                    vm = (lane + sg * L) < q
                    rows = lane + sg * L

                    def d_body(d, _):
                        dsp = jnp.full((L,), 1, jnp.int32) * d
                        vals = plsc.load_gather(src, [dsp, o16], mask=vm)
                        plsc.store_scatter(rowbuf, [rows, dsp], vals, mask=vm)
                        return 0

                    lax.fori_loop(0, D, d_body, 0, unroll=8)
                    plsc.store_scatter(bidx, [rows], b16, mask=vm)

            pltpu.async_copy(rowbuf, out_hbm.at[bidx], sem_w)

        def sweep(c0, c1, sbase, src):
            def rs_cond(st):
                gi, q = st
                return gi < ngr

            def rs_body(st):
                gi, q = st
                b16 = blist[pl.ds(pl.multiple_of(gi * L, L), L)]
                vm = (lane + gi * L) < cnt
                hid = plsc.load_gather(ids_v, [b16], mask=vm)
                m = vm & (hid >= c0) & (hid < c1)
                mi = m.astype(jnp.int32)
                pos = q + plsc.cumsum(mi) - mi
                plsc.store_scatter(cb, [pos], b16, mask=m)
                plsc.store_scatter(co, [pos], hid - sbase, mask=m)
                q = q + jnp.sum(mi)

                @pl.when(q > G - L)
                def _():
                    fire(q, src)

                return gi + 1, jnp.where(q > G - L, 0, q)

            gi, q = lax.while_loop(rs_cond, rs_body, (jnp.int32(0), jnp.int32(0)))

            @pl.when(q > 0)
            def _():
                fire(q, src)

        def sbase_of(c):
            c0 = lo + c * CW
            return pl.multiple_of(jnp.minimum(c0, PHYS - CW), 128)

        def issue(c, stg, sem):
            s = sbase_of(c)
            for i in range(D // 8):
                pltpu.async_copy(
                    tT_hbm.at[pl.ds(8 * i, 8), pl.ds(s, CW)],
                    stg.at[pl.ds(8 * i, 8), :],
                    sem,
                )

        def drain_stage(stg, sem):
            for i in range(D // 8):
                pltpu.make_async_copy(
                    tT_hbm.at[pl.ds(0, 8), pl.ds(0, CW)],
                    stg.at[pl.ds(8 * i, 8), :],
                    sem,
                ).wait()

        def do_chunk(c, stg):
            c0 = lo + c * CW
            c1 = jnp.minimum(c0 + CW, hi)
            sweep(c0, c1, sbase_of(c), stg)

        issue(0, stage0, sem_s0)

        def pair_body(p, _):
            c = p * 2
            drain_stage(stage0, sem_s0)

            @pl.when(c + 1 < NCH)
            def _():
                issue(c + 1, stage1, sem_s1)

            do_chunk(c, stage0)

            @pl.when(c + 1 < NCH)
            def _():
                drain_stage(stage1, sem_s1)

                @pl.when(c + 2 < NCH)
                def _():
                    issue(c + 2, stage0, sem_s0)

                do_chunk(c + 1, stage1)

            return 0

        lax.fori_loop(0, (NCH + 1) // 2, pair_body, 0, unroll=False)
        drain_scatter()

    out_full = emb(tableT, node_ids.astype(jnp.int32))
    return out_full[:B, :D]
